# Initial kernel scaffold; baseline (speedup 1.0000x reference)
import jax
import jax.numpy as jnp
from jax.experimental import pallas as pl


def kernel(x, Wg, W1, W2, W3):
    raise NotImplementedError("write your pallas kernel here")


# speedup vs baseline: 1.7366x; 1.7366x over previous
import functools

import jax
import jax.numpy as jnp
from jax import lax
from jax.experimental import pallas as pl
from jax.experimental.pallas import tpu as pltpu

DIM = 768
HID = 2048
E = 8
K = 2
HC = 512
NH = HID // HC


def _moe_body(x_ref, wg_ref, w1_ref, w3_ref, w2_ref, o_ref, w_scr, *, T):
    e = pl.program_id(0)
    h = pl.program_id(1)

    @pl.when(jnp.logical_and(e == 0, h == 0))
    def _init():
        o_ref[...] = jnp.zeros_like(o_ref)

    @pl.when(h == 0)
    def _gate():
        xb = x_ref[...]
        logits = lax.dot_general(xb, wg_ref[...], (((1,), (1,)), ((), ())),
                                 preferred_element_type=jnp.float32)
        m = jnp.max(logits, axis=-1, keepdims=True)
        p = jnp.exp(logits - m)
        p = p / jnp.sum(p, axis=-1, keepdims=True)
        iota = lax.broadcasted_iota(jnp.int32, (T, E), 1)
        m1 = jnp.max(p, axis=-1, keepdims=True)
        i1 = jnp.min(jnp.where(p == m1, iota, E), axis=-1, keepdims=True)
        sel1 = iota == i1
        p2 = jnp.where(sel1, -jnp.inf, p)
        m2 = jnp.max(p2, axis=-1, keepdims=True)
        i2 = jnp.min(jnp.where(p2 == m2, iota, E), axis=-1, keepdims=True)
        sel2 = iota == i2
        denom = m1 + m2 + 1e-20
        w_all = (jnp.where(sel1, p, 0.0) + jnp.where(sel2, p, 0.0)) / denom
        w_scr[...] = jnp.sum(jnp.where(iota == e, w_all, 0.0), axis=-1,
                             keepdims=True)

    xb = x_ref[...]
    w1 = w1_ref[0]
    w3 = w3_ref[0]
    w2 = w2_ref[0]
    h1 = lax.dot_general(xb, w1, (((1,), (1,)), ((), ())),
                         preferred_element_type=jnp.float32)
    h3 = lax.dot_general(xb, w3, (((1,), (1,)), ((), ())),
                         preferred_element_type=jnp.float32)
    hid = (h1 * jax.nn.sigmoid(h1)) * h3
    out = lax.dot_general(hid, w2, (((1,), (1,)), ((), ())),
                          preferred_element_type=jnp.float32)
    o_ref[...] += w_scr[...] * out


def kernel(x, Wg, W1, W2, W3):
    b, s, d = x.shape
    T = b * s
    flat = x.reshape(T, d)
    out = pl.pallas_call(
        functools.partial(_moe_body, T=T),
        grid=(E, NH),
        in_specs=[
            pl.BlockSpec((T, DIM), lambda e, h: (0, 0)),
            pl.BlockSpec((E, DIM), lambda e, h: (0, 0)),
            pl.BlockSpec((1, HC, DIM), lambda e, h: (e, h, 0)),
            pl.BlockSpec((1, HC, DIM), lambda e, h: (e, h, 0)),
            pl.BlockSpec((1, DIM, HC), lambda e, h: (e, 0, h)),
        ],
        out_specs=pl.BlockSpec((T, DIM), lambda e, h: (0, 0)),
        out_shape=jax.ShapeDtypeStruct((T, DIM), jnp.float32),
        scratch_shapes=[pltpu.VMEM((T, 1), jnp.float32)],
    )(flat, Wg, W1, W3, W2)
    return out.reshape(b, s, d)
